# baseline (speedup 1.0000x reference)
import functools

import jax
import jax.numpy as jnp
from jax.experimental import pallas as pl
from jax.experimental.pallas import tpu as pltpu


_ROW_BLOCK = 1024


def _topk_sum(x, k):
    bits = jax.lax.bitcast_convert_type(x, jnp.int32)

    def bs(_, lohi):
        lo, hi = lohi
        mid = lo + (hi - lo + 1) // 2
        cnt = jnp.sum((bits >= mid).astype(jnp.int32))
        take = cnt >= k
        return jnp.where(take, mid, lo), jnp.where(take, hi, mid - 1)

    lo, _ = jax.lax.fori_loop(
        0, 32, bs, (jnp.int32(0), jnp.int32(0x7F000000)))
    v = jax.lax.bitcast_convert_type(lo, jnp.float32)
    sum_gt = jnp.sum(jnp.where(x > v, x, 0.0))
    cnt_gt = jnp.sum((x > v).astype(jnp.float32))
    return sum_gt + (jnp.float32(k) - cnt_gt) * v


def _loss_kernel(xa_ref, ya_ref, out_ref, row_ref, *, n, k):
    blk = min(_ROW_BLOCK, n)
    ya = ya_ref[0]

    def step(i, acc2):
        xb = xa_ref[0, pl.ds(i * blk, blk), :]
        t = jax.lax.dot_general(
            xb, ya, (((1,), (1,)), ((), ())),
            preferred_element_type=jnp.float32,
        )
        row_ref[pl.ds(i * blk, blk), :] = jnp.min(
            t.reshape(blk, n // 128, 128), axis=1)
        return jnp.minimum(acc2, jnp.min(t, axis=0, keepdims=True))

    acc0 = jnp.full((1, n), jnp.inf, dtype=jnp.float32)
    m2 = jax.lax.fori_loop(0, n // blk, step, acc0)
    m1 = jnp.min(row_ref[:, :].T, axis=0, keepdims=True)
    dist2 = jnp.sqrt(jnp.maximum(m2, 0.0))
    dist1 = jnp.sqrt(jnp.maximum(m1, 0.0))
    inv_n = jnp.float32(1.0 / n)
    loss_cd = (jnp.sum(dist1) + jnp.sum(dist2)) * inv_n
    loss_w = (_topk_sum(dist1, k) + _topk_sum(dist2, k)) * jnp.float32(1.0 / k)
    out_ref[0, 0, :] = jnp.full((128,), loss_cd + 3.0 * loss_w, jnp.float32)


def _hi_lo(x2):
    hi_f = jax.lax.bitcast_convert_type(
        jax.lax.bitcast_convert_type(x2, jnp.int32) & jnp.int32(-65536),
        jnp.float32)
    return hi_f.astype(jnp.bfloat16), (x2 - hi_f).astype(jnp.bfloat16)


def kernel(pred_pointclouds, gt_pointclouds):
    pred = pred_pointclouds.astype(jnp.float32)
    gt = gt_pointclouds.astype(jnp.float32)
    b, n, _ = pred.shape
    k = int(0.5 * n)

    p2 = jnp.sum(pred * pred, axis=-1, keepdims=True)
    g2 = jnp.sum(gt * gt, axis=-1, keepdims=True)
    p2hi, p2lo = _hi_lo(p2)
    g2hi, g2lo = _hi_lo(g2)
    ones = jnp.ones((b, n, 1), jnp.bfloat16)
    zpad = jnp.zeros((b, n, 1), jnp.bfloat16)
    xa = jnp.concatenate(
        [-2.0 * pred.astype(jnp.bfloat16), p2hi, p2lo, ones, ones, zpad],
        axis=-1)
    ya = jnp.concatenate(
        [gt.astype(jnp.bfloat16), ones, ones, g2hi, g2lo, zpad],
        axis=-1)

    spec = pl.BlockSpec((1, n, 8), lambda i: (i, 0, 0))
    losses = pl.pallas_call(
        functools.partial(_loss_kernel, n=n, k=k),
        grid=(b,),
        in_specs=[spec, spec],
        out_specs=pl.BlockSpec((1, 1, 128), lambda i: (i, 0, 0)),
        out_shape=jax.ShapeDtypeStruct((b, 1, 128), jnp.float32),
        scratch_shapes=[pltpu.VMEM((n, 128), jnp.float32)],
    )(xa, ya)
    return jnp.sum(losses[:, 0, 0]) / b
